# initial kernel scaffold (unmeasured)
import jax
import jax.numpy as jnp
from jax import lax
from jax.experimental import pallas as pl
from jax.experimental.pallas import tpu as pltpu

N_EXPERTS = 4
EXPERTS_PER_SHARD = 2


def kernel(x, assign, W1, W2):
    tokens, d_model = x.shape
    n_loc, _, d_ff = W1.shape

    my_x = lax.axis_index("x")
    onehot = (assign[:, None] == jnp.arange(N_EXPERTS, dtype=assign.dtype)[None, :]).astype(x.dtype)
    m_self = lax.dynamic_slice(onehot, (0, EXPERTS_PER_SHARD * my_x), (tokens, EXPERTS_PER_SHARD))
    m_out = lax.dynamic_slice(onehot, (0, EXPERTS_PER_SHARD * (1 - my_x)), (tokens, EXPERTS_PER_SHARD))

    def body(x_ref, m_self_ref, m_out_ref, w1_ref, w2_ref, out_ref,
             x_peer, m_peer, res_out, res_in, send_sems, recv_sems):
        mx = lax.axis_index("x")
        my = lax.axis_index("y")
        mz = lax.axis_index("z")
        peer = (1 - mx, my, mz)

        barrier_sem = pltpu.get_barrier_semaphore()
        pl.semaphore_signal(barrier_sem, inc=1, device_id=peer,
                            device_id_type=pl.DeviceIdType.MESH)
        pl.semaphore_wait(barrier_sem, 1)

        rdma_x = pltpu.make_async_remote_copy(
            src_ref=x_ref, dst_ref=x_peer,
            send_sem=send_sems.at[0], recv_sem=recv_sems.at[0],
            device_id=peer, device_id_type=pl.DeviceIdType.MESH)
        rdma_x.start()
        rdma_m = pltpu.make_async_remote_copy(
            src_ref=m_out_ref, dst_ref=m_peer,
            send_sem=send_sems.at[1], recv_sem=recv_sems.at[1],
            device_id=peer, device_id_type=pl.DeviceIdType.MESH)
        rdma_m.start()

        xl = x_ref[...]
        for k in range(EXPERTS_PER_SHARD):
            h = jnp.maximum(jnp.dot(xl, w1_ref[k], preferred_element_type=jnp.float32), 0.0)
            p = jnp.dot(h, w2_ref[k], preferred_element_type=jnp.float32)
            masked = p * m_self_ref[:, k:k + 1]
            if k == 0:
                out_ref[...] = masked
            else:
                out_ref[...] = out_ref[...] + masked

        rdma_x.wait()
        rdma_m.wait()

        xp = x_peer[...]
        for k in range(EXPERTS_PER_SHARD):
            h = jnp.maximum(jnp.dot(xp, w1_ref[k], preferred_element_type=jnp.float32), 0.0)
            p = jnp.dot(h, w2_ref[k], preferred_element_type=jnp.float32)
            masked = p * m_peer[:, k:k + 1]
            if k == 0:
                res_out[...] = masked
            else:
                res_out[...] = res_out[...] + masked

        rdma_r = pltpu.make_async_remote_copy(
            src_ref=res_out, dst_ref=res_in,
            send_sem=send_sems.at[2], recv_sem=recv_sems.at[2],
            device_id=peer, device_id_type=pl.DeviceIdType.MESH)
        rdma_r.start()
        rdma_r.wait()

        out_ref[...] = out_ref[...] + res_in[...]

    return pl.pallas_call(
        body,
        out_shape=jax.ShapeDtypeStruct((tokens, d_model), x.dtype),
        in_specs=[pl.BlockSpec(memory_space=pltpu.VMEM)] * 5,
        out_specs=pl.BlockSpec(memory_space=pltpu.VMEM),
        scratch_shapes=[
            pltpu.VMEM((tokens, d_model), x.dtype),
            pltpu.VMEM((tokens, EXPERTS_PER_SHARD), x.dtype),
            pltpu.VMEM((tokens, d_model), x.dtype),
            pltpu.VMEM((tokens, d_model), x.dtype),
            pltpu.SemaphoreType.DMA((3,)),
            pltpu.SemaphoreType.DMA((3,)),
        ],
        compiler_params=pltpu.CompilerParams(collective_id=0),
    )(x, m_self, m_out, W1, W2)


# baseline (device time: 140410 ns/iter reference)
import jax
import jax.numpy as jnp
from jax import lax
from jax.experimental import pallas as pl
from jax.experimental.pallas import tpu as pltpu

N_EXPERTS = 4
EXPERTS_PER_SHARD = 2


def kernel(x, assign, W1, W2):
    tokens, d_model = x.shape
    n_loc, _, d_ff = W1.shape

    my_x = lax.axis_index("x")
    onehot = (assign[:, None] == jnp.arange(N_EXPERTS, dtype=assign.dtype)[None, :]).astype(x.dtype)
    m_self = lax.dynamic_slice(onehot, (0, EXPERTS_PER_SHARD * my_x), (tokens, EXPERTS_PER_SHARD))
    m_out = lax.dynamic_slice(onehot, (0, EXPERTS_PER_SHARD * (1 - my_x)), (tokens, EXPERTS_PER_SHARD))

    def body(x_ref, m_self_ref, m_out_ref, w1_ref, w2_ref, out_ref,
             x_peer, m_peer, res_out, res_in, send_sems, recv_sems):
        mx = lax.axis_index("x")
        my = lax.axis_index("y")
        mz = lax.axis_index("z")
        peer = (1 - mx, my, mz)

        barrier_sem = pltpu.get_barrier_semaphore()
        pl.semaphore_signal(barrier_sem, inc=1, device_id=peer,
                            device_id_type=pl.DeviceIdType.MESH)
        pl.semaphore_wait(barrier_sem, 1)

        rdma_x = pltpu.make_async_remote_copy(
            src_ref=x_ref, dst_ref=x_peer,
            send_sem=send_sems.at[0], recv_sem=recv_sems.at[0],
            device_id=peer, device_id_type=pl.DeviceIdType.MESH)
        rdma_x.start()
        rdma_m = pltpu.make_async_remote_copy(
            src_ref=m_out_ref, dst_ref=m_peer,
            send_sem=send_sems.at[1], recv_sem=recv_sems.at[1],
            device_id=peer, device_id_type=pl.DeviceIdType.MESH)
        rdma_m.start()

        xl = x_ref[...]
        for k in range(EXPERTS_PER_SHARD):
            h = jnp.maximum(jnp.dot(xl, w1_ref[k], preferred_element_type=jnp.float32), 0.0)
            p = jnp.dot(h, w2_ref[k], preferred_element_type=jnp.float32)
            masked = p * m_self_ref[:, k:k + 1]
            if k == 0:
                out_ref[...] = masked
            else:
                out_ref[...] = out_ref[...] + masked

        rdma_x.wait()
        rdma_m.wait()

        xp = x_peer[...]
        for k in range(EXPERTS_PER_SHARD):
            h = jnp.maximum(jnp.dot(xp, w1_ref[k], preferred_element_type=jnp.float32), 0.0)
            p = jnp.dot(h, w2_ref[k], preferred_element_type=jnp.float32)
            masked = p * m_peer[:, k:k + 1]
            if k == 0:
                res_out[...] = masked
            else:
                res_out[...] = res_out[...] + masked

        rdma_r = pltpu.make_async_remote_copy(
            src_ref=res_out, dst_ref=res_in,
            send_sem=send_sems.at[2], recv_sem=recv_sems.at[2],
            device_id=peer, device_id_type=pl.DeviceIdType.MESH)
        rdma_r.start()
        rdma_r.wait()

        out_ref[...] = out_ref[...] + res_in[...]

    return pl.pallas_call(
        body,
        out_shape=jax.ShapeDtypeStruct((tokens, d_model), x.dtype),
        in_specs=[pl.BlockSpec(memory_space=pltpu.VMEM)] * 5,
        out_specs=pl.BlockSpec(memory_space=pltpu.VMEM),
        scratch_shapes=[
            pltpu.VMEM((tokens, d_model), x.dtype),
            pltpu.VMEM((tokens, EXPERTS_PER_SHARD), x.dtype),
            pltpu.VMEM((tokens, d_model), x.dtype),
            pltpu.VMEM((tokens, d_model), x.dtype),
            pltpu.SemaphoreType.DMA((3,)),
            pltpu.SemaphoreType.DMA((3,)),
        ],
        compiler_params=pltpu.CompilerParams(
            collective_id=0,
            vmem_limit_bytes=100 * 1024 * 1024,
        ),
    )(x, m_self, m_out, W1, W2)


# device time: 128289 ns/iter; 1.0945x vs baseline; 1.0945x over previous
import jax
import jax.numpy as jnp
from jax import lax
from jax.experimental import pallas as pl
from jax.experimental.pallas import tpu as pltpu

N_EXPERTS = 4
EXPERTS_PER_SHARD = 2
N_CHUNKS = 4


def kernel(x, assign, W1, W2):
    tokens, d_model = x.shape
    n_loc, _, d_ff = W1.shape

    my_x = lax.axis_index("x")
    onehot = (assign[:, None] == jnp.arange(N_EXPERTS, dtype=assign.dtype)[None, :]).astype(x.dtype)
    m_self = lax.dynamic_slice(onehot, (0, EXPERTS_PER_SHARD * my_x), (tokens, EXPERTS_PER_SHARD))
    m_out = lax.dynamic_slice(onehot, (0, EXPERTS_PER_SHARD * (1 - my_x)), (tokens, EXPERTS_PER_SHARD))

    def body(x_ref, m_self_ref, m_out_ref, w1_ref, w2_ref, out_ref,
             x_peer, m_peer, res_out, res_in, send_sems, recv_sems):
        mx = lax.axis_index("x")
        my = lax.axis_index("y")
        mz = lax.axis_index("z")
        peer = (1 - mx, my, mz)

        barrier_sem = pltpu.get_barrier_semaphore()
        pl.semaphore_signal(barrier_sem, inc=1, device_id=peer,
                            device_id_type=pl.DeviceIdType.MESH)
        pl.semaphore_wait(barrier_sem, 1)

        rdma_x = pltpu.make_async_remote_copy(
            src_ref=x_ref, dst_ref=x_peer,
            send_sem=send_sems.at[0], recv_sem=recv_sems.at[0],
            device_id=peer, device_id_type=pl.DeviceIdType.MESH)
        rdma_x.start()
        rdma_m = pltpu.make_async_remote_copy(
            src_ref=m_out_ref, dst_ref=m_peer,
            send_sem=send_sems.at[1], recv_sem=recv_sems.at[1],
            device_id=peer, device_id_type=pl.DeviceIdType.MESH)
        rdma_m.start()

        xl = x_ref[...]
        for k in range(EXPERTS_PER_SHARD):
            h = jnp.maximum(jnp.dot(xl, w1_ref[k], preferred_element_type=jnp.float32), 0.0)
            p = jnp.dot(h, w2_ref[k], preferred_element_type=jnp.float32)
            masked = p * m_self_ref[:, k:k + 1]
            if k == 0:
                out_ref[...] = masked
            else:
                out_ref[...] = out_ref[...] + masked

        rdma_x.wait()
        rdma_m.wait()

        chunk = tokens // N_CHUNKS
        res_rdmas = []
        for c in range(N_CHUNKS):
            rows = pl.ds(c * chunk, chunk)
            xp = x_peer[rows, :]
            for k in range(EXPERTS_PER_SHARD):
                h = jnp.maximum(jnp.dot(xp, w1_ref[k], preferred_element_type=jnp.float32), 0.0)
                p = jnp.dot(h, w2_ref[k], preferred_element_type=jnp.float32)
                masked = p * m_peer[rows, k:k + 1]
                if k == 0:
                    res_out[rows, :] = masked
                else:
                    res_out[rows, :] = res_out[rows, :] + masked
            rdma_r = pltpu.make_async_remote_copy(
                src_ref=res_out.at[rows, :], dst_ref=res_in.at[rows, :],
                send_sem=send_sems.at[2 + c], recv_sem=recv_sems.at[2 + c],
                device_id=peer, device_id_type=pl.DeviceIdType.MESH)
            rdma_r.start()
            res_rdmas.append(rdma_r)

        for rdma_r in res_rdmas:
            rdma_r.wait()

        out_ref[...] = out_ref[...] + res_in[...]

    return pl.pallas_call(
        body,
        out_shape=jax.ShapeDtypeStruct((tokens, d_model), x.dtype),
        in_specs=[pl.BlockSpec(memory_space=pltpu.VMEM)] * 5,
        out_specs=pl.BlockSpec(memory_space=pltpu.VMEM),
        scratch_shapes=[
            pltpu.VMEM((tokens, d_model), x.dtype),
            pltpu.VMEM((tokens, EXPERTS_PER_SHARD), x.dtype),
            pltpu.VMEM((tokens, d_model), x.dtype),
            pltpu.VMEM((tokens, d_model), x.dtype),
            pltpu.SemaphoreType.DMA((2 + N_CHUNKS,)),
            pltpu.SemaphoreType.DMA((2 + N_CHUNKS,)),
        ],
        compiler_params=pltpu.CompilerParams(
            collective_id=0,
            vmem_limit_bytes=100 * 1024 * 1024,
        ),
    )(x, m_self, m_out, W1, W2)
